# Initial kernel scaffold; baseline (speedup 1.0000x reference)
import jax
import jax.numpy as jnp
from jax.experimental import pallas as pl


def kernel(inputs_rgb, inputs_ir, targets_rgb, targets_ir, features):
    raise NotImplementedError("write your pallas kernel here")


# speedup vs baseline: 1.9442x; 1.9442x over previous
import jax
import jax.numpy as jnp
from jax.experimental import pallas as pl
from jax.experimental.pallas import tpu as pltpu

_B = 1024
_D = 64
_M = 100000
_TEMP = 0.05
_TILE_M = 800
_STEPS = _M // _TILE_M
_B2 = 2 * _B


def _body(x_ref, t_ref, f_ref, out_rgb_ref, out_ir_ref,
          xn_ref, acc_s_ref, acc_ll_ref):
    i = pl.program_id(0)

    @pl.when(i == 0)
    def _init():
        x = x_ref[...]
        n = jnp.sqrt(jnp.sum(x * x, axis=1, keepdims=True))
        xn_ref[...] = x / jnp.maximum(n, 1e-12)
        acc_s_ref[...] = jnp.zeros_like(acc_s_ref)
        acc_ll_ref[...] = jnp.zeros_like(acc_ll_ref)

    f = f_ref[...]
    xn = xn_ref[...]
    logits = jax.lax.dot_general(
        f, xn, (((1,), (1,)), ((), ())),
        preferred_element_type=jnp.float32) * (1.0 / _TEMP)
    e = jnp.exp(logits)
    acc_s_ref[...] += jnp.sum(e, axis=0, keepdims=True)
    gid = i * _TILE_M + jax.lax.broadcasted_iota(jnp.int32, (_TILE_M, _B2), 0)
    hit = gid == t_ref[...]
    acc_ll_ref[...] += jnp.sum(jnp.where(hit, logits, 0.0), axis=0,
                               keepdims=True)

    @pl.when(i == _STEPS - 1)
    def _fini():
        diff = jnp.log(acc_s_ref[...]) - acc_ll_ref[...]
        out_rgb_ref[...] = jnp.mean(diff[:, :_B], axis=1, keepdims=True)
        out_ir_ref[...] = jnp.mean(diff[:, _B:], axis=1, keepdims=True)


def kernel(inputs_rgb, inputs_ir, targets_rgb, targets_ir, features):
    x = jnp.concatenate([inputs_rgb, inputs_ir], axis=0)
    t = jnp.concatenate([targets_rgb, targets_ir], axis=0)
    t = t.astype(jnp.int32).reshape(1, _B2)
    out_rgb, out_ir = pl.pallas_call(
        _body,
        grid=(_STEPS,),
        in_specs=[
            pl.BlockSpec((_B2, _D), lambda i: (0, 0)),
            pl.BlockSpec((1, _B2), lambda i: (0, 0)),
            pl.BlockSpec((_TILE_M, _D), lambda i: (i, 0)),
        ],
        out_specs=[
            pl.BlockSpec((1, 1), lambda i: (0, 0)),
            pl.BlockSpec((1, 1), lambda i: (0, 0)),
        ],
        out_shape=[
            jax.ShapeDtypeStruct((1, 1), jnp.float32),
            jax.ShapeDtypeStruct((1, 1), jnp.float32),
        ],
        scratch_shapes=[
            pltpu.VMEM((_B2, _D), jnp.float32),
            pltpu.VMEM((1, _B2), jnp.float32),
            pltpu.VMEM((1, _B2), jnp.float32),
        ],
        compiler_params=pltpu.CompilerParams(
            dimension_semantics=("arbitrary",)),
    )(x, t, features)
    return (out_rgb[0, 0], out_ir[0, 0])


# speedup vs baseline: 2.3046x; 1.1854x over previous
import functools
import math

import jax
import jax.numpy as jnp
from jax import lax
from jax.experimental import pallas as pl
from jax.experimental.pallas import tpu as pltpu
from jax.experimental.pallas import tpu_sc as plsc

_B = 1024
_D = 64
_M = 100000
_TEMP = 0.05
_TILE_M = 800
_STEPS = _M // _TILE_M
_B2 = 2 * _B
_SCALE = math.log2(math.e) / _TEMP
_LN2 = math.log(2.0)

_NC = 2
_NS = 16
_NW = _NC * _NS
_BPW = _B2 // _NW


@functools.cache
def _make_gather():
    @functools.partial(
        pl.kernel,
        mesh=plsc.VectorSubcoreMesh(core_axis_name="c",
                                    subcore_axis_name="s"),
        out_type=jax.ShapeDtypeStruct((_B2, _D), jnp.float32),
        scratch_types=[
            pltpu.VMEM((_BPW,), jnp.int32),
            pltpu.VMEM((_BPW, _D), jnp.float32),
            pltpu.SemaphoreType.DMA,
        ],
        compiler_params=pltpu.CompilerParams(use_tc_tiling_on_sc=False),
    )
    def _gather(table_hbm, idx_hbm, out_hbm, idx_v, rows_v, sem):
        wid = lax.axis_index("s") * _NC + lax.axis_index("c")
        base = wid * _BPW
        pltpu.sync_copy(idx_hbm.at[pl.ds(base, _BPW)], idx_v)
        pltpu.async_copy(table_hbm.at[idx_v], rows_v, sem).wait()
        pltpu.sync_copy(rows_v, out_hbm.at[pl.ds(base, _BPW)])

    return _gather


def _gather_rows(table, idx):
    return _make_gather()(table, idx)


def _body(x_ref, f_ref, g_ref, out_rgb_ref, out_ir_ref, xn_ref, acc_ref):
    i = pl.program_id(0)

    @pl.when(i == 0)
    def _init():
        x = x_ref[...]
        n = jnp.sqrt(jnp.sum(x * x, axis=1, keepdims=True))
        xn_ref[...] = x * (_SCALE / jnp.maximum(n, 1e-12))
        acc_ref[...] = jnp.zeros_like(acc_ref)

    logits2 = jax.lax.dot_general(
        f_ref[...], xn_ref[...], (((1,), (1,)), ((), ())),
        preferred_element_type=jnp.float32)
    acc_ref[...] += jnp.sum(jnp.exp2(logits2), axis=0, keepdims=True)

    @pl.when(i == _STEPS - 1)
    def _fini():
        logz = jnp.log(acc_ref[...])
        ll2 = jnp.sum(xn_ref[...] * g_ref[...], axis=1, keepdims=True)
        out_rgb_ref[...] = (jnp.mean(logz[:, :_B], axis=1, keepdims=True)
                            - _LN2 * jnp.mean(ll2[:_B, :], axis=0,
                                              keepdims=True))
        out_ir_ref[...] = (jnp.mean(logz[:, _B:], axis=1, keepdims=True)
                           - _LN2 * jnp.mean(ll2[_B:, :], axis=0,
                                             keepdims=True))


def kernel(inputs_rgb, inputs_ir, targets_rgb, targets_ir, features):
    x = jnp.concatenate([inputs_rgb, inputs_ir], axis=0)
    t = jnp.concatenate([targets_rgb, targets_ir], axis=0).astype(jnp.int32)
    g = _gather_rows(features, t)
    out_rgb, out_ir = pl.pallas_call(
        _body,
        grid=(_STEPS,),
        in_specs=[
            pl.BlockSpec((_B2, _D), lambda i: (0, 0)),
            pl.BlockSpec((_TILE_M, _D), lambda i: (i, 0)),
            pl.BlockSpec((_B2, _D), lambda i: (0, 0)),
        ],
        out_specs=[
            pl.BlockSpec((1, 1), lambda i: (0, 0)),
            pl.BlockSpec((1, 1), lambda i: (0, 0)),
        ],
        out_shape=[
            jax.ShapeDtypeStruct((1, 1), jnp.float32),
            jax.ShapeDtypeStruct((1, 1), jnp.float32),
        ],
        scratch_shapes=[
            pltpu.VMEM((_B2, _D), jnp.float32),
            pltpu.VMEM((1, _B2), jnp.float32),
        ],
        compiler_params=pltpu.CompilerParams(
            dimension_semantics=("arbitrary",)),
    )(x, features, g)
    return (out_rgb[0, 0], out_ir[0, 0])


# speedup vs baseline: 2.7259x; 1.1828x over previous
import functools
import math

import jax
import jax.numpy as jnp
from jax import lax
from jax.experimental import pallas as pl
from jax.experimental.pallas import tpu as pltpu
from jax.experimental.pallas import tpu_sc as plsc

_B = 1024
_D = 64
_M = 100000
_TEMP = 0.05
_TILE_M = 800
_STEPS = _M // _TILE_M
_B2 = 2 * _B
_SCALE = math.log2(math.e) / _TEMP
_LN2 = math.log(2.0)

_NC = 2
_NS = 16
_NW = _NC * _NS
_BPW = _B2 // _NW


@functools.cache
def _make_gather():
    @functools.partial(
        pl.kernel,
        mesh=plsc.VectorSubcoreMesh(core_axis_name="c",
                                    subcore_axis_name="s"),
        out_type=jax.ShapeDtypeStruct((_B2, _D), jnp.float32),
        scratch_types=[
            pltpu.VMEM((_BPW,), jnp.int32),
            pltpu.VMEM((_BPW, _D), jnp.float32),
            pltpu.SemaphoreType.DMA,
        ],
        compiler_params=pltpu.CompilerParams(use_tc_tiling_on_sc=False),
    )
    def _gather(table_hbm, idx_hbm, out_hbm, idx_v, rows_v, sem):
        wid = lax.axis_index("s") * _NC + lax.axis_index("c")
        base = wid * _BPW
        pltpu.sync_copy(idx_hbm.at[pl.ds(base, _BPW)], idx_v)
        pltpu.async_copy(table_hbm.at[idx_v], rows_v, sem).wait()
        pltpu.sync_copy(rows_v, out_hbm.at[pl.ds(base, _BPW)])

    return _gather


def _gather_rows(table, idx):
    return _make_gather()(table, idx)


def _body(x_ref, f_ref, g_ref, out_rgb_ref, out_ir_ref, xn_ref, acc_ref):
    i = pl.program_id(0)

    @pl.when(i == 0)
    def _init():
        x = x_ref[...]
        n = jnp.sqrt(jnp.sum(x * x, axis=1, keepdims=True))
        xn_ref[...] = x * (_SCALE / jnp.maximum(n, 1e-12))
        acc_ref[...] = jnp.zeros_like(acc_ref)

    logits2 = jax.lax.dot_general(
        f_ref[...], xn_ref[...], (((1,), (1,)), ((), ())),
        preferred_element_type=jnp.float32)
    acc_ref[...] += jnp.sum(jnp.exp2(logits2), axis=0, keepdims=True)

    @pl.when(i == _STEPS - 1)
    def _fini():
        logz = jnp.log(acc_ref[...])
        ll2 = jnp.sum(xn_ref[...] * g_ref[...], axis=1, keepdims=True)
        out_rgb_ref[...] = (jnp.mean(logz[:, :_B], axis=1, keepdims=True)
                            - _LN2 * jnp.mean(ll2[:_B, :], axis=0,
                                              keepdims=True))
        out_ir_ref[...] = (jnp.mean(logz[:, _B:], axis=1, keepdims=True)
                           - _LN2 * jnp.mean(ll2[_B:, :], axis=0,
                                             keepdims=True))


def kernel(inputs_rgb, inputs_ir, targets_rgb, targets_ir, features):
    x = jnp.concatenate([inputs_rgb, inputs_ir], axis=0)
    t = jnp.concatenate([targets_rgb, targets_ir], axis=0).astype(jnp.int32)
    g = jnp.take(features, t, axis=0)
    out_rgb, out_ir = pl.pallas_call(
        _body,
        grid=(_STEPS,),
        in_specs=[
            pl.BlockSpec((_B2, _D), lambda i: (0, 0)),
            pl.BlockSpec((_TILE_M, _D), lambda i: (i, 0)),
            pl.BlockSpec((_B2, _D), lambda i: (0, 0)),
        ],
        out_specs=[
            pl.BlockSpec((1, 1), lambda i: (0, 0)),
            pl.BlockSpec((1, 1), lambda i: (0, 0)),
        ],
        out_shape=[
            jax.ShapeDtypeStruct((1, 1), jnp.float32),
            jax.ShapeDtypeStruct((1, 1), jnp.float32),
        ],
        scratch_shapes=[
            pltpu.VMEM((_B2, _D), jnp.float32),
            pltpu.VMEM((1, _B2), jnp.float32),
        ],
        compiler_params=pltpu.CompilerParams(
            dimension_semantics=("arbitrary",)),
    )(x, features, g)
    return (out_rgb[0, 0], out_ir[0, 0])


# speedup vs baseline: 2.9589x; 1.0855x over previous
import functools
import math

import jax
import jax.numpy as jnp
from jax import lax
from jax.experimental import pallas as pl
from jax.experimental.pallas import tpu as pltpu
from jax.experimental.pallas import tpu_sc as plsc

_B = 1024
_D = 64
_M = 100000
_TEMP = 0.05
_TILE_M = 2000
_STEPS = _M // _TILE_M
_B2 = 2 * _B
_SCALE = math.log2(math.e) / _TEMP
_LN2 = math.log(2.0)


def _main_body(x_ref, f_ref, mz_rgb_ref, mz_ir_ref, xn_ref, acc_ref):
    i = pl.program_id(0)

    @pl.when(i == 0)
    def _init():
        x = x_ref[...]
        n = jnp.sqrt(jnp.sum(x * x, axis=1, keepdims=True))
        xn_ref[...] = x * (_SCALE / jnp.maximum(n, 1e-12))
        acc_ref[...] = jnp.zeros_like(acc_ref)

    logits2 = jax.lax.dot_general(
        f_ref[...], xn_ref[...], (((1,), (1,)), ((), ())),
        preferred_element_type=jnp.float32)
    acc_ref[...] += jnp.sum(jnp.exp2(logits2), axis=0, keepdims=True)

    @pl.when(i == _STEPS - 1)
    def _fini():
        logz = jnp.log(acc_ref[...])
        mz_rgb_ref[...] = jnp.mean(logz[:, :_B], axis=1, keepdims=True)
        mz_ir_ref[...] = jnp.mean(logz[:, _B:], axis=1, keepdims=True)


def _join_body(x_ref, g_ref, mz_rgb_ref, mz_ir_ref, out_rgb_ref, out_ir_ref):
    x = x_ref[...]
    n = jnp.sqrt(jnp.sum(x * x, axis=1, keepdims=True))
    xn = x / jnp.maximum(n, 1e-12)
    ll = jnp.sum(xn * g_ref[...], axis=1, keepdims=True) * (1.0 / _TEMP)
    out_rgb_ref[...] = mz_rgb_ref[...] - jnp.mean(ll[:_B, :], axis=0,
                                                  keepdims=True)
    out_ir_ref[...] = mz_ir_ref[...] - jnp.mean(ll[_B:, :], axis=0,
                                                keepdims=True)


def kernel(inputs_rgb, inputs_ir, targets_rgb, targets_ir, features):
    x = jnp.concatenate([inputs_rgb, inputs_ir], axis=0)
    t = jnp.concatenate([targets_rgb, targets_ir], axis=0).astype(jnp.int32)
    g = jnp.take(features, t, axis=0)
    mz_rgb, mz_ir = pl.pallas_call(
        _main_body,
        grid=(_STEPS,),
        in_specs=[
            pl.BlockSpec((_B2, _D), lambda i: (0, 0)),
            pl.BlockSpec((_TILE_M, _D), lambda i: (i, 0)),
        ],
        out_specs=[
            pl.BlockSpec((1, 1), lambda i: (0, 0)),
            pl.BlockSpec((1, 1), lambda i: (0, 0)),
        ],
        out_shape=[
            jax.ShapeDtypeStruct((1, 1), jnp.float32),
            jax.ShapeDtypeStruct((1, 1), jnp.float32),
        ],
        scratch_shapes=[
            pltpu.VMEM((_B2, _D), jnp.float32),
            pltpu.VMEM((1, _B2), jnp.float32),
        ],
        compiler_params=pltpu.CompilerParams(
            dimension_semantics=("arbitrary",)),
    )(x, features)
    out_rgb, out_ir = pl.pallas_call(
        _join_body,
        out_shape=[
            jax.ShapeDtypeStruct((1, 1), jnp.float32),
            jax.ShapeDtypeStruct((1, 1), jnp.float32),
        ],
    )(x, g, mz_rgb, mz_ir)
    return (out_rgb[0, 0], out_ir[0, 0])
